# Initial kernel scaffold; baseline (speedup 1.0000x reference)
import jax
import jax.numpy as jnp
from jax.experimental import pallas as pl


def kernel(x, edge_index, W_lin, b_lin, W_beta, b_beta):
    raise NotImplementedError("write your pallas kernel here")


# speedup vs baseline: 10.1610x; 10.1610x over previous
import functools

import jax
import jax.numpy as jnp
from jax import lax
from jax.experimental import pallas as pl
from jax.experimental.pallas import tpu as pltpu
from jax.experimental.pallas import tpu_sc as plsc

MINN = 1e-15
MAXNORM = 1.0 - 4e-3
NC, NS = 2, 16
CHUNK = 128


def _artanh(v):
    v = jnp.clip(v, -1.0 + 1e-15, 1.0 - 1e-15)
    return 0.5 * (jnp.log1p(v) - jnp.log1p(-v))


def _rnorm(v):
    return jnp.maximum(jnp.sqrt(jnp.sum(v * v, axis=-1, keepdims=True)), MINN)


def _proj(v):
    nrm = _rnorm(v)
    return jnp.where(nrm > MAXNORM, v / nrm * MAXNORM, v)


def _expmap0(u):
    un = _rnorm(u)
    return jnp.tanh(un) * u / un


def _logmap0(p):
    pn = _rnorm(p)
    return _artanh(pn) / pn * p


def _tc1_body(x_ref, w_ref, b_ref, hrow_ref, hself_ref,
              h_ref, h1_ref, a1_ref, a2_ref):
    x = x_ref[...]
    W = w_ref[...]
    xn = _rnorm(x)
    mx = lax.dot_general(x, W, (((1,), (1,)), ((), ())),
                         preferred_element_type=jnp.float32)
    mxn = _rnorm(mx)
    res_c = jnp.tanh(mxn / xn * _artanh(xn)) * mx / mxn
    res_c = jnp.where(jnp.all(mx == 0, axis=-1, keepdims=True), 0.0, res_c)
    res = _proj(res_c)
    hyp_bias = _proj(_expmap0(b_ref[...]))
    x2 = jnp.sum(res * res, axis=-1, keepdims=True)
    y2 = jnp.sum(hyp_bias * hyp_bias, axis=-1, keepdims=True)
    xy = jnp.sum(res * hyp_bias, axis=-1, keepdims=True)
    num = (1.0 + 2.0 * xy + y2) * res + (1.0 - x2) * hyp_bias
    den = jnp.maximum(1.0 + 2.0 * xy + x2 * y2, MINN)
    h = _logmap0(_proj(num / den))
    h_ref[...] = h
    deg = hrow_ref[...] - hself_ref[...]
    dinv = jnp.where(deg > 0, lax.rsqrt(deg), 0.0)
    h1_ref[...] = dinv * h
    a1_ref[...] = dinv
    a2_ref[...] = hself_ref[...] * dinv


def _tc2_body(n_real, k, h_ref, h1_ref, a1_ref, a2_ref, s1a_ref, s1b_ref,
              t2_ref, sel_ref):
    h = h_ref[...]
    info = h - a1_ref[...] * s1a_ref[...] + a2_ref[...] * h1_ref[...]
    score = jnp.sum(jnp.abs(info), axis=-1, keepdims=True)
    bits = lax.bitcast_convert_type(score, jnp.int32)
    rid = lax.broadcasted_iota(jnp.int32, bits.shape, 0)
    bits = jnp.where(rid < n_real, bits, -1)

    def body(_, lohi):
        lo, hi = lohi
        mid = lo + (hi - lo + 1) // 2
        ok = jnp.sum((bits >= mid).astype(jnp.int32)) >= k
        return jnp.where(ok, mid, lo), jnp.where(ok, hi, mid - 1)

    lo, _ = lax.fori_loop(0, 31, body,
                          (jnp.int32(0), jnp.int32(0x7F800000)))
    sel = (bits > lo).astype(jnp.float32)
    sel_ref[...] = sel
    t2_ref[...] = sel * h


def _tc3_body(s1b_ref, s2a_ref, s2b_ref, h_ref, sel_ref, wb1_ref, wb2_ref,
              bb_ref, t3_ref):
    u1 = s2a_ref[...] + s2b_ref[...]
    u2 = s1b_ref[...]
    sq = lambda a, b: jnp.maximum(
        jnp.sqrt(jnp.sum(a * a, axis=-1, keepdims=True)
                 + jnp.sum(b * b, axis=-1, keepdims=True)), MINN)
    un = sq(u1, u2)
    sc_e = jnp.tanh(un) / un
    p1, p2 = sc_e * u1, sc_e * u2
    pn = sq(p1, p2)
    f = jnp.where(pn > MAXNORM, MAXNORM / pn, 1.0)
    hp1, hp2 = f * p1, f * p2
    xn = sq(hp1, hp2)
    mx = (jnp.sum(hp1 * wb1_ref[...], axis=-1, keepdims=True)
          + jnp.sum(hp2 * wb2_ref[...], axis=-1, keepdims=True))
    mxn = jnp.maximum(jnp.abs(mx), MINN)
    res_c = jnp.tanh(mxn / xn * _artanh(xn)) * mx / mxn
    res_c = jnp.where(mx == 0, 0.0, res_c)

    def proj1(v):
        vn = jnp.maximum(jnp.abs(v), MINN)
        return jnp.where(vn > MAXNORM, v / vn * MAXNORM, v)

    res = proj1(res_c)
    bb = bb_ref[...]
    bn = jnp.maximum(jnp.abs(bb), MINN)
    hyp_bias = proj1(jnp.tanh(bn) * bb / bn)
    x2 = res * res
    y2 = hyp_bias * hyp_bias
    xy = res * hyp_bias
    num = (1.0 + 2.0 * xy + y2) * res + (1.0 - x2) * hyp_bias
    den = jnp.maximum(1.0 + 2.0 * xy + x2 * y2, MINN)
    beta_out = proj1(num / den)
    bon = jnp.maximum(jnp.abs(beta_out), MINN)
    wlog = _artanh(bon) / bon * beta_out
    weight = 1.0 / (1.0 + jnp.exp(-wlog))
    t3_ref[...] = weight * sel_ref[...] * h_ref[...]


def _tc4_body(h_ref, s3a_ref, s3b_ref, out_ref):
    a = jnp.maximum(s3a_ref[...] + s3b_ref[...], 0.0)
    out_ref[...] = _proj(_expmap0(h_ref[...] + a))


def _tc_call(body, out_shapes):
    return pl.pallas_call(body, out_shape=out_shapes)


def _sc_hist(nph, n_chunk_rows):
    cpt = n_chunk_rows // NS
    zr = nph // NS
    mesh = plsc.VectorSubcoreMesh(core_axis_name="c", subcore_axis_name="s",
                                  num_cores=NC, num_subcores=NS)

    zoffs = list(range(0, zr - CHUNK, CHUNK)) + [zr - CHUNK]

    @functools.partial(
        pl.kernel, mesh=mesh,
        out_type=(jax.ShapeDtypeStruct((nph,), jnp.float32),
                  jax.ShapeDtypeStruct((nph,), jnp.float32)),
        scratch_types=[
            pltpu.VMEM((cpt, CHUNK), jnp.int32),
            pltpu.VMEM((CHUNK,), jnp.float32),
            pltpu.VMEM((CHUNK,), jnp.float32),
            pltpu.VMEM((nph // NS,), jnp.float32),
            pltpu.VMEM_SHARED((nph,), jnp.float32),
        ],
    )
    def hist_kernel(idxa_hbm, idxb_hbm, ones_hbm, zeros_hbm, outa_hbm,
                    outb_hbm, idx_v, ones_v, zero_v, wb_v, acc_sh):
        cid = lax.axis_index("c")
        sid = lax.axis_index("s")
        pltpu.sync_copy(zeros_hbm, zero_v)
        for off in zoffs:
            pltpu.sync_copy(zero_v, acc_sh.at[pl.ds(sid * zr + off, CHUNK)])
        pltpu.sync_copy(ones_hbm, ones_v)

        @pl.when(cid == 0)
        def _():
            pltpu.sync_copy(idxa_hbm.at[pl.ds(sid * cpt, cpt)], idx_v)

        @pl.when(cid == 1)
        def _():
            pltpu.sync_copy(idxb_hbm.at[pl.ds(sid * cpt, cpt)], idx_v)

        plsc.subcore_barrier()

        def step(j, carry):
            pltpu.sync_copy(ones_v, acc_sh.at[idx_v.at[j]], add=True)
            return carry

        lax.fori_loop(0, cpt, step, 0)
        plsc.subcore_barrier()
        pltpu.sync_copy(acc_sh.at[pl.ds(sid * zr, zr)], wb_v)

        @pl.when(cid == 0)
        def _():
            pltpu.sync_copy(wb_v, outa_hbm.at[pl.ds(sid * zr, zr)])

        @pl.when(cid == 1)
        def _():
            pltpu.sync_copy(wb_v, outb_hbm.at[pl.ds(sid * zr, zr)])

    return hist_kernel


def _sc_prop(np_rows, d, n_chunk_rows, split_edges):
    cpt = n_chunk_rows // (NS * NC if split_edges else NS)
    zr = np_rows // NS
    zoffs = list(range(0, zr - CHUNK, CHUNK)) + [zr - CHUNK]
    mesh = plsc.VectorSubcoreMesh(core_axis_name="c", subcore_axis_name="s",
                                  num_cores=NC, num_subcores=NS)

    grp = 16
    n_grp = cpt // grp

    @functools.partial(
        pl.kernel, mesh=mesh,
        out_type=jax.ShapeDtypeStruct((NC, np_rows, d), jnp.float32),
        scratch_types=[
            pltpu.VMEM((grp, CHUNK), jnp.int32),
            pltpu.VMEM((grp, CHUNK), jnp.int32),
            pltpu.VMEM((CHUNK, d), jnp.float32),
            pltpu.VMEM_SHARED((np_rows, d), jnp.float32),
            pltpu.SemaphoreType.DMA,
        ],
    )
    def prop_kernel(tab_hbm, rowa_hbm, rowb_hbm, col_hbm, zeros_hbm, out_hbm,
                    idx_v, col_v, gbuf, acc_sh, sem):
        cid = lax.axis_index("c")
        sid = lax.axis_index("s")
        pltpu.sync_copy(zeros_hbm, gbuf)
        for off in zoffs:
            pltpu.sync_copy(gbuf, acc_sh.at[pl.ds(sid * zr + off, CHUNK)])
        if split_edges:
            base = (sid * NC + cid) * cpt
        else:
            base = sid * cpt
        plsc.subcore_barrier()

        def group(g, carry):
            gb = base + g * grp

            @pl.when(cid == 0)
            def _():
                pltpu.sync_copy(rowa_hbm.at[pl.ds(gb, grp)], idx_v)

            @pl.when(cid == 1)
            def _():
                pltpu.sync_copy(rowb_hbm.at[pl.ds(gb, grp)], idx_v)

            pltpu.sync_copy(col_hbm.at[pl.ds(gb, grp)], col_v)

            def step(j, c2):
                pltpu.async_copy(tab_hbm.at[idx_v.at[j]], gbuf, sem).wait()
                pltpu.sync_copy(gbuf, acc_sh.at[col_v.at[j]], add=True)
                return c2

            lax.fori_loop(0, grp, step, carry)
            return carry

        lax.fori_loop(0, n_grp, group, 0)
        plsc.subcore_barrier()
        wb_blocks = [(i * CHUNK, CHUNK) for i in range(zr // CHUNK)]
        if zr % CHUNK:
            wb_blocks.append((zr // CHUNK * CHUNK, zr % CHUNK))
        for off, rows in wb_blocks:
            pltpu.sync_copy(acc_sh.at[pl.ds(sid * zr + off, rows)],
                            gbuf.at[pl.ds(0, rows)])
            pltpu.sync_copy(gbuf.at[pl.ds(0, rows)],
                            out_hbm.at[cid, pl.ds(sid * zr + off, rows)])

    return prop_kernel


def kernel(x, edge_index, W_lin, b_lin, W_beta, b_beta):
    n, d = x.shape
    e = edge_index.shape[1]
    np_rows = -(-n // (NS * 8)) * (NS * 8)
    ep = -(-e // (NC * NS * CHUNK * 8)) * (NC * NS * CHUNK * 8)
    npad = ep - e
    nph = -(-(np_rows + 4096) // (NS * 8)) * (NS * 8)

    row = edge_index[0]
    col = edge_index[1]
    pad_nodes = n + (jnp.arange(npad, dtype=jnp.int32) % (np_rows - n))
    row_p = jnp.concatenate([row, pad_nodes])
    col_p = jnp.concatenate([col, pad_nodes])
    dump = np_rows + (jnp.arange(ep, dtype=jnp.int32) % 4096)
    self_p = jnp.where(row_p == col_p, row_p, dump)
    ncr = ep // CHUNK
    row2d = row_p.reshape(ncr, CHUNK)
    col2d = col_p.reshape(ncr, CHUNK)
    self2d = self_p.reshape(ncr, CHUNK)
    rowb2d = row2d + np_rows

    ones128 = jnp.ones((CHUNK,), jnp.float32)
    zeros_h = jnp.zeros((CHUNK,), jnp.float32)
    zeros_nd = jnp.zeros((CHUNK, d), jnp.float32)
    x_pad = jnp.concatenate([x, jnp.zeros((np_rows - n, d), x.dtype)])

    hista, histb = _sc_hist(nph, ncr)(row2d, self2d, ones128, zeros_h)
    hrow = hista[:np_rows].reshape(np_rows, 1)
    hself = histb[:np_rows].reshape(np_rows, 1)

    nd = jax.ShapeDtypeStruct((np_rows, d), jnp.float32)
    n1 = jax.ShapeDtypeStruct((np_rows, 1), jnp.float32)
    h, h1, a1, a2 = _tc_call(_tc1_body, (nd, nd, n1, n1))(
        x_pad, W_lin, b_lin.reshape(1, d), hrow, hself)

    tab1 = jnp.concatenate([h1, h], axis=0)
    s1 = _sc_prop(np_rows, d, ncr, split_edges=False)(
        tab1, row2d, rowb2d, col2d, zeros_nd)

    k = int(n * 0.75)
    t2, sel = _tc_call(functools.partial(_tc2_body, n, k), (nd, n1))(
        h, h1, a1, a2, s1[0], s1[1])

    prop_split = _sc_prop(np_rows, d, ncr, split_edges=True)
    s2 = prop_split(t2, row2d, row2d, col2d, zeros_nd)

    t3 = _tc_call(_tc3_body, nd)(
        s1[1], s2[0], s2[1], h, sel,
        W_beta[:, :d], W_beta[:, d:], b_beta.reshape(1, 1))

    s3 = prop_split(t3, row2d, row2d, col2d, zeros_nd)

    out = _tc_call(_tc4_body, nd)(h, s3[0], s3[1])
    return out[:n]


# speedup vs baseline: 12.4926x; 1.2295x over previous
import functools

import jax
import jax.numpy as jnp
from jax import lax
from jax.experimental import pallas as pl
from jax.experimental.pallas import tpu as pltpu
from jax.experimental.pallas import tpu_sc as plsc

MINN = 1e-15
MAXNORM = 1.0 - 4e-3
NC, NS = 2, 16
CHUNK = 128


def _artanh(v):
    v = jnp.clip(v, -1.0 + 1e-15, 1.0 - 1e-15)
    return 0.5 * (jnp.log1p(v) - jnp.log1p(-v))


def _rnorm(v):
    return jnp.maximum(jnp.sqrt(jnp.sum(v * v, axis=-1, keepdims=True)), MINN)


def _proj(v):
    nrm = _rnorm(v)
    return jnp.where(nrm > MAXNORM, v / nrm * MAXNORM, v)


def _expmap0(u):
    un = _rnorm(u)
    return jnp.tanh(un) * u / un


def _logmap0(p):
    pn = _rnorm(p)
    return _artanh(pn) / pn * p


def _tc1_body(x_ref, w_ref, b_ref, hrow_ref, hself_ref,
              h_ref, h1_ref, a1_ref, a2_ref):
    x = x_ref[...]
    W = w_ref[...]
    xn = _rnorm(x)
    mx = lax.dot_general(x, W, (((1,), (1,)), ((), ())),
                         preferred_element_type=jnp.float32)
    mxn = _rnorm(mx)
    res_c = jnp.tanh(mxn / xn * _artanh(xn)) * mx / mxn
    res_c = jnp.where(jnp.all(mx == 0, axis=-1, keepdims=True), 0.0, res_c)
    res = _proj(res_c)
    hyp_bias = _proj(_expmap0(b_ref[...]))
    x2 = jnp.sum(res * res, axis=-1, keepdims=True)
    y2 = jnp.sum(hyp_bias * hyp_bias, axis=-1, keepdims=True)
    xy = jnp.sum(res * hyp_bias, axis=-1, keepdims=True)
    num = (1.0 + 2.0 * xy + y2) * res + (1.0 - x2) * hyp_bias
    den = jnp.maximum(1.0 + 2.0 * xy + x2 * y2, MINN)
    h = _logmap0(_proj(num / den))
    h_ref[...] = h
    deg = hrow_ref[...] - hself_ref[...]
    dinv = jnp.where(deg > 0, lax.rsqrt(deg), 0.0)
    h1_ref[...] = dinv * h
    a1_ref[...] = dinv
    a2_ref[...] = hself_ref[...] * dinv


def _tc2_body(n_real, k, h_ref, h1_ref, a1_ref, a2_ref, s1a_ref, s1b_ref,
              t2_ref, sel_ref):
    h = h_ref[...]
    info = h - a1_ref[...] * s1a_ref[...] + a2_ref[...] * h1_ref[...]
    score = jnp.sum(jnp.abs(info), axis=-1, keepdims=True)
    bits = lax.bitcast_convert_type(score, jnp.int32)
    rid = lax.broadcasted_iota(jnp.int32, bits.shape, 0)
    bits = jnp.where(rid < n_real, bits, -1)

    def body(_, lohi):
        lo, hi = lohi
        mid = lo + (hi - lo + 1) // 2
        ok = jnp.sum((bits >= mid).astype(jnp.int32)) >= k
        return jnp.where(ok, mid, lo), jnp.where(ok, hi, mid - 1)

    lo, _ = lax.fori_loop(0, 31, body,
                          (jnp.int32(0), jnp.int32(0x7F800000)))
    sel = (bits > lo).astype(jnp.float32)
    sel_ref[...] = sel
    t2_ref[...] = sel * h


def _tc3_body(s1b_ref, s2a_ref, s2b_ref, h_ref, sel_ref, wb1_ref, wb2_ref,
              bb_ref, t3_ref):
    u1 = s2a_ref[...] + s2b_ref[...]
    u2 = s1b_ref[...]
    sq = lambda a, b: jnp.maximum(
        jnp.sqrt(jnp.sum(a * a, axis=-1, keepdims=True)
                 + jnp.sum(b * b, axis=-1, keepdims=True)), MINN)
    un = sq(u1, u2)
    sc_e = jnp.tanh(un) / un
    p1, p2 = sc_e * u1, sc_e * u2
    pn = sq(p1, p2)
    f = jnp.where(pn > MAXNORM, MAXNORM / pn, 1.0)
    hp1, hp2 = f * p1, f * p2
    xn = sq(hp1, hp2)
    mx = (jnp.sum(hp1 * wb1_ref[...], axis=-1, keepdims=True)
          + jnp.sum(hp2 * wb2_ref[...], axis=-1, keepdims=True))
    mxn = jnp.maximum(jnp.abs(mx), MINN)
    res_c = jnp.tanh(mxn / xn * _artanh(xn)) * mx / mxn
    res_c = jnp.where(mx == 0, 0.0, res_c)

    def proj1(v):
        vn = jnp.maximum(jnp.abs(v), MINN)
        return jnp.where(vn > MAXNORM, v / vn * MAXNORM, v)

    res = proj1(res_c)
    bb = bb_ref[...]
    bn = jnp.maximum(jnp.abs(bb), MINN)
    hyp_bias = proj1(jnp.tanh(bn) * bb / bn)
    x2 = res * res
    y2 = hyp_bias * hyp_bias
    xy = res * hyp_bias
    num = (1.0 + 2.0 * xy + y2) * res + (1.0 - x2) * hyp_bias
    den = jnp.maximum(1.0 + 2.0 * xy + x2 * y2, MINN)
    beta_out = proj1(num / den)
    bon = jnp.maximum(jnp.abs(beta_out), MINN)
    wlog = _artanh(bon) / bon * beta_out
    weight = 1.0 / (1.0 + jnp.exp(-wlog))
    t3_ref[...] = weight * sel_ref[...] * h_ref[...]


def _tc4_body(h_ref, s3a_ref, s3b_ref, out_ref):
    a = jnp.maximum(s3a_ref[...] + s3b_ref[...], 0.0)
    out_ref[...] = _proj(_expmap0(h_ref[...] + a))


def _tc_call(body, out_shapes):
    return pl.pallas_call(body, out_shape=out_shapes)


def _sc_hist(nph, n_chunk_rows):
    cpt = n_chunk_rows // NS
    zr = nph // NS
    mesh = plsc.VectorSubcoreMesh(core_axis_name="c", subcore_axis_name="s",
                                  num_cores=NC, num_subcores=NS)

    zoffs = list(range(0, zr - CHUNK, CHUNK)) + [zr - CHUNK]

    @functools.partial(
        pl.kernel, mesh=mesh,
        out_type=(jax.ShapeDtypeStruct((nph,), jnp.float32),
                  jax.ShapeDtypeStruct((nph,), jnp.float32)),
        scratch_types=[
            pltpu.VMEM((cpt, CHUNK), jnp.int32),
            pltpu.VMEM((CHUNK,), jnp.float32),
            pltpu.VMEM((CHUNK,), jnp.float32),
            pltpu.VMEM((nph // NS,), jnp.float32),
            pltpu.VMEM_SHARED((nph,), jnp.float32),
        ],
    )
    def hist_kernel(idxa_hbm, idxb_hbm, ones_hbm, zeros_hbm, outa_hbm,
                    outb_hbm, idx_v, ones_v, zero_v, wb_v, acc_sh):
        cid = lax.axis_index("c")
        sid = lax.axis_index("s")
        pltpu.sync_copy(zeros_hbm, zero_v)
        for off in zoffs:
            pltpu.sync_copy(zero_v, acc_sh.at[pl.ds(sid * zr + off, CHUNK)])
        pltpu.sync_copy(ones_hbm, ones_v)

        @pl.when(cid == 0)
        def _():
            pltpu.sync_copy(idxa_hbm.at[pl.ds(sid * cpt, cpt)], idx_v)

        @pl.when(cid == 1)
        def _():
            pltpu.sync_copy(idxb_hbm.at[pl.ds(sid * cpt, cpt)], idx_v)

        plsc.subcore_barrier()

        def step(j, carry):
            pltpu.sync_copy(ones_v, acc_sh.at[idx_v.at[j]], add=True)
            return carry

        lax.fori_loop(0, cpt, step, 0)
        plsc.subcore_barrier()
        pltpu.sync_copy(acc_sh.at[pl.ds(sid * zr, zr)], wb_v)

        @pl.when(cid == 0)
        def _():
            pltpu.sync_copy(wb_v, outa_hbm.at[pl.ds(sid * zr, zr)])

        @pl.when(cid == 1)
        def _():
            pltpu.sync_copy(wb_v, outb_hbm.at[pl.ds(sid * zr, zr)])

    return hist_kernel


def _sc_prop(np_rows, d, n_chunk_rows, split_edges):
    cpt = n_chunk_rows // (NS * NC if split_edges else NS)
    zr = np_rows // NS
    zoffs = list(range(0, zr - CHUNK, CHUNK)) + [zr - CHUNK]
    mesh = plsc.VectorSubcoreMesh(core_axis_name="c", subcore_axis_name="s",
                                  num_cores=NC, num_subcores=NS)

    grp = 16
    n_grp = cpt // grp

    @functools.partial(
        pl.kernel, mesh=mesh,
        out_type=jax.ShapeDtypeStruct((NC, np_rows, d), jnp.float32),
        scratch_types=[
            pltpu.VMEM((grp, CHUNK), jnp.int32),
            pltpu.VMEM((grp, CHUNK), jnp.int32),
            pltpu.VMEM((2, CHUNK, d), jnp.float32),
            pltpu.VMEM_SHARED((np_rows, d), jnp.float32),
            pltpu.SemaphoreType.DMA((2,)),
        ],
    )
    def prop_kernel(tab_hbm, rowa_hbm, rowb_hbm, col_hbm, zeros_hbm, out_hbm,
                    idx_v, col_v, gbuf, acc_sh, sem):
        cid = lax.axis_index("c")
        sid = lax.axis_index("s")
        pltpu.sync_copy(zeros_hbm, gbuf.at[0])
        for off in zoffs:
            pltpu.sync_copy(gbuf.at[0], acc_sh.at[pl.ds(sid * zr + off, CHUNK)])
        if split_edges:
            base = (sid * NC + cid) * cpt
        else:
            base = sid * cpt
        plsc.subcore_barrier()

        def group(g, carry):
            gb = base + g * grp

            @pl.when(cid == 0)
            def _():
                pltpu.sync_copy(rowa_hbm.at[pl.ds(gb, grp)], idx_v)

            @pl.when(cid == 1)
            def _():
                pltpu.sync_copy(rowb_hbm.at[pl.ds(gb, grp)], idx_v)

            pltpu.sync_copy(col_hbm.at[pl.ds(gb, grp)], col_v)
            pltpu.async_copy(tab_hbm.at[idx_v.at[0]], gbuf.at[0], sem.at[0])
            for j in range(grp):
                b = j % 2
                pltpu.make_async_copy(tab_hbm.at[idx_v.at[j]],
                                      gbuf.at[b], sem.at[b]).wait()
                if j + 1 < grp:
                    pltpu.async_copy(tab_hbm.at[idx_v.at[j + 1]],
                                     gbuf.at[1 - b], sem.at[1 - b])
                pltpu.sync_copy(gbuf.at[b], acc_sh.at[col_v.at[j]], add=True)
            return carry

        lax.fori_loop(0, n_grp, group, 0)
        plsc.subcore_barrier()
        wb_blocks = [(i * CHUNK, CHUNK) for i in range(zr // CHUNK)]
        if zr % CHUNK:
            wb_blocks.append((zr // CHUNK * CHUNK, zr % CHUNK))
        for off, rows in wb_blocks:
            pltpu.sync_copy(acc_sh.at[pl.ds(sid * zr + off, rows)],
                            gbuf.at[0, pl.ds(0, rows)])
            pltpu.sync_copy(gbuf.at[0, pl.ds(0, rows)],
                            out_hbm.at[cid, pl.ds(sid * zr + off, rows)])

    return prop_kernel


def kernel(x, edge_index, W_lin, b_lin, W_beta, b_beta):
    n, d = x.shape
    e = edge_index.shape[1]
    np_rows = -(-n // (NS * 8)) * (NS * 8)
    ep = -(-e // (NC * NS * CHUNK * 8)) * (NC * NS * CHUNK * 8)
    npad = ep - e
    nph = -(-(np_rows + 4096) // (NS * 8)) * (NS * 8)

    row = edge_index[0]
    col = edge_index[1]
    pad_nodes = n + (jnp.arange(npad, dtype=jnp.int32) % (np_rows - n))
    row_p = jnp.concatenate([row, pad_nodes])
    col_p = jnp.concatenate([col, pad_nodes])
    dump = np_rows + (jnp.arange(ep, dtype=jnp.int32) % 4096)
    self_p = jnp.where(row_p == col_p, row_p, dump)
    ncr = ep // CHUNK
    row2d = row_p.reshape(ncr, CHUNK)
    col2d = col_p.reshape(ncr, CHUNK)
    self2d = self_p.reshape(ncr, CHUNK)
    rowb2d = row2d + np_rows

    ones128 = jnp.ones((CHUNK,), jnp.float32)
    zeros_h = jnp.zeros((CHUNK,), jnp.float32)
    zeros_nd = jnp.zeros((CHUNK, d), jnp.float32)
    x_pad = jnp.concatenate([x, jnp.zeros((np_rows - n, d), x.dtype)])

    hista, histb = _sc_hist(nph, ncr)(row2d, self2d, ones128, zeros_h)
    hrow = hista[:np_rows].reshape(np_rows, 1)
    hself = histb[:np_rows].reshape(np_rows, 1)

    nd = jax.ShapeDtypeStruct((np_rows, d), jnp.float32)
    n1 = jax.ShapeDtypeStruct((np_rows, 1), jnp.float32)
    h, h1, a1, a2 = _tc_call(_tc1_body, (nd, nd, n1, n1))(
        x_pad, W_lin, b_lin.reshape(1, d), hrow, hself)

    tab1 = jnp.concatenate([h1, h], axis=0)
    s1 = _sc_prop(np_rows, d, ncr, split_edges=False)(
        tab1, row2d, rowb2d, col2d, zeros_nd)

    k = int(n * 0.75)
    t2, sel = _tc_call(functools.partial(_tc2_body, n, k), (nd, n1))(
        h, h1, a1, a2, s1[0], s1[1])

    prop_split = _sc_prop(np_rows, d, ncr, split_edges=True)
    s2 = prop_split(t2, row2d, row2d, col2d, zeros_nd)

    t3 = _tc_call(_tc3_body, nd)(
        s1[1], s2[0], s2[1], h, sel,
        W_beta[:, :d], W_beta[:, d:], b_beta.reshape(1, 1))

    s3 = prop_split(t3, row2d, row2d, col2d, zeros_nd)

    out = _tc_call(_tc4_body, nd)(h, s3[0], s3[1])
    return out[:n]


# speedup vs baseline: 13.1467x; 1.0524x over previous
import functools

import jax
import jax.numpy as jnp
from jax import lax
from jax.experimental import pallas as pl
from jax.experimental.pallas import tpu as pltpu
from jax.experimental.pallas import tpu_sc as plsc

MINN = 1e-15
MAXNORM = 1.0 - 4e-3
NC, NS = 2, 16
CHUNK = 128


def _artanh(v):
    v = jnp.clip(v, -1.0 + 1e-15, 1.0 - 1e-15)
    return 0.5 * (jnp.log1p(v) - jnp.log1p(-v))


def _rnorm(v):
    return jnp.maximum(jnp.sqrt(jnp.sum(v * v, axis=-1, keepdims=True)), MINN)


def _proj(v):
    nrm = _rnorm(v)
    return jnp.where(nrm > MAXNORM, v / nrm * MAXNORM, v)


def _expmap0(u):
    un = _rnorm(u)
    return jnp.tanh(un) * u / un


def _logmap0(p):
    pn = _rnorm(p)
    return _artanh(pn) / pn * p


def _tc1a_body(x_ref, w_ref, b_ref, h_ref):
    x = x_ref[...]
    W = w_ref[...]
    xn = _rnorm(x)
    mx = lax.dot_general(x, W, (((1,), (1,)), ((), ())),
                         preferred_element_type=jnp.float32)
    mxn = _rnorm(mx)
    res_c = jnp.tanh(mxn / xn * _artanh(xn)) * mx / mxn
    res_c = jnp.where(jnp.all(mx == 0, axis=-1, keepdims=True), 0.0, res_c)
    res = _proj(res_c)
    hyp_bias = _proj(_expmap0(b_ref[...]))
    x2 = jnp.sum(res * res, axis=-1, keepdims=True)
    y2 = jnp.sum(hyp_bias * hyp_bias, axis=-1, keepdims=True)
    xy = jnp.sum(res * hyp_bias, axis=-1, keepdims=True)
    num = (1.0 + 2.0 * xy + y2) * res + (1.0 - x2) * hyp_bias
    den = jnp.maximum(1.0 + 2.0 * xy + x2 * y2, MINN)
    h_ref[...] = _logmap0(_proj(num / den))


def _tc1b_body(np_rows, h_ref, hrow_ref, hself_ref, tab_ref, a1_ref, a2_ref):
    h = h_ref[...]
    deg = hrow_ref[...] - hself_ref[...]
    dinv = jnp.where(deg > 0, lax.rsqrt(deg), 0.0)
    tab_ref[0:np_rows, :] = dinv * h
    tab_ref[np_rows:2 * np_rows, :] = h
    a1_ref[...] = dinv
    a2_ref[...] = hself_ref[...] * dinv


def _tc2_body(np_rows, n_real, k, tab_ref, a1_ref, a2_ref, s1_ref,
              t2_ref, sel_ref):
    h1 = tab_ref[0:np_rows, :]
    h = tab_ref[np_rows:2 * np_rows, :]
    info = h - a1_ref[...] * s1_ref[0] + a2_ref[...] * h1
    score = jnp.sum(jnp.abs(info), axis=-1, keepdims=True)
    bits = lax.bitcast_convert_type(score, jnp.int32)
    rid = lax.broadcasted_iota(jnp.int32, bits.shape, 0)
    bits = jnp.where(rid < n_real, bits, -1)
    bflat = jnp.reshape(bits, (np_rows // 128, 128))

    def body(_, lohi):
        lo, hi = lohi
        mid = lo + (hi - lo + 1) // 2
        ok = jnp.sum((bflat >= mid).astype(jnp.int32)) >= k
        return jnp.where(ok, mid, lo), jnp.where(ok, hi, mid - 1)

    lo, _ = lax.fori_loop(0, 31, body,
                          (jnp.int32(0), jnp.int32(0x7F800000)))
    sel = (bits > lo).astype(jnp.float32)
    sel_ref[...] = sel
    t2_ref[...] = sel * h


def _tc3_body(s1_ref, s2_ref, h_ref, sel_ref, wb1_ref, wb2_ref,
              bb_ref, t3_ref):
    u1 = s2_ref[0] + s2_ref[1]
    u2 = s1_ref[1]
    sq = lambda a, b: jnp.maximum(
        jnp.sqrt(jnp.sum(a * a, axis=-1, keepdims=True)
                 + jnp.sum(b * b, axis=-1, keepdims=True)), MINN)
    un = sq(u1, u2)
    sc_e = jnp.tanh(un) / un
    p1, p2 = sc_e * u1, sc_e * u2
    pn = sq(p1, p2)
    f = jnp.where(pn > MAXNORM, MAXNORM / pn, 1.0)
    hp1, hp2 = f * p1, f * p2
    xn = sq(hp1, hp2)
    mx = (jnp.sum(hp1 * wb1_ref[...], axis=-1, keepdims=True)
          + jnp.sum(hp2 * wb2_ref[...], axis=-1, keepdims=True))
    mxn = jnp.maximum(jnp.abs(mx), MINN)
    res_c = jnp.tanh(mxn / xn * _artanh(xn)) * mx / mxn
    res_c = jnp.where(mx == 0, 0.0, res_c)

    def proj1(v):
        vn = jnp.maximum(jnp.abs(v), MINN)
        return jnp.where(vn > MAXNORM, v / vn * MAXNORM, v)

    res = proj1(res_c)
    bb = bb_ref[...]
    bn = jnp.maximum(jnp.abs(bb), MINN)
    hyp_bias = proj1(jnp.tanh(bn) * bb / bn)
    x2 = res * res
    y2 = hyp_bias * hyp_bias
    xy = res * hyp_bias
    num = (1.0 + 2.0 * xy + y2) * res + (1.0 - x2) * hyp_bias
    den = jnp.maximum(1.0 + 2.0 * xy + x2 * y2, MINN)
    beta_out = proj1(num / den)
    bon = jnp.maximum(jnp.abs(beta_out), MINN)
    wlog = _artanh(bon) / bon * beta_out
    weight = 1.0 / (1.0 + jnp.exp(-wlog))
    t3_ref[...] = weight * sel_ref[...] * h_ref[...]


def _tc4_body(h_ref, s3_ref, out_ref):
    a = jnp.maximum(s3_ref[0] + s3_ref[1], 0.0)
    out_ref[...] = _proj(_expmap0(h_ref[...] + a))


def _tc_call(body, out_shapes):
    return pl.pallas_call(body, out_shape=out_shapes)


def _sc_hist(nph, n_chunk_rows):
    cpt = n_chunk_rows // NS
    zr = nph // NS
    mesh = plsc.VectorSubcoreMesh(core_axis_name="c", subcore_axis_name="s",
                                  num_cores=NC, num_subcores=NS)

    zoffs = list(range(0, zr - CHUNK, CHUNK)) + [zr - CHUNK]

    @functools.partial(
        pl.kernel, mesh=mesh,
        out_type=(jax.ShapeDtypeStruct((nph,), jnp.float32),
                  jax.ShapeDtypeStruct((nph,), jnp.float32)),
        scratch_types=[
            pltpu.VMEM((cpt, CHUNK), jnp.int32),
            pltpu.VMEM((CHUNK,), jnp.float32),
            pltpu.VMEM((CHUNK,), jnp.float32),
            pltpu.VMEM((nph // NS,), jnp.float32),
            pltpu.VMEM_SHARED((nph,), jnp.float32),
        ],
    )
    def hist_kernel(idxa_hbm, idxb_hbm, ones_hbm, zeros_hbm, outa_hbm,
                    outb_hbm, idx_v, ones_v, zero_v, wb_v, acc_sh):
        cid = lax.axis_index("c")
        sid = lax.axis_index("s")
        pltpu.sync_copy(zeros_hbm, zero_v)
        for off in zoffs:
            pltpu.sync_copy(zero_v, acc_sh.at[pl.ds(sid * zr + off, CHUNK)])
        pltpu.sync_copy(ones_hbm, ones_v)

        @pl.when(cid == 0)
        def _():
            pltpu.sync_copy(idxa_hbm.at[pl.ds(sid * cpt, cpt)], idx_v)

        @pl.when(cid == 1)
        def _():
            pltpu.sync_copy(idxb_hbm.at[pl.ds(sid * cpt, cpt)], idx_v)

        plsc.subcore_barrier()

        def step(j, carry):
            pltpu.sync_copy(ones_v, acc_sh.at[idx_v.at[j]], add=True)
            return carry

        lax.fori_loop(0, cpt, step, 0)
        plsc.subcore_barrier()
        pltpu.sync_copy(acc_sh.at[pl.ds(sid * zr, zr)], wb_v)

        @pl.when(cid == 0)
        def _():
            pltpu.sync_copy(wb_v, outa_hbm.at[pl.ds(sid * zr, zr)])

        @pl.when(cid == 1)
        def _():
            pltpu.sync_copy(wb_v, outb_hbm.at[pl.ds(sid * zr, zr)])

    return hist_kernel


def _sc_prop(np_rows, d, n_chunk_rows, split_edges):
    cpt = n_chunk_rows // (NS * NC if split_edges else NS)
    zr = np_rows // NS
    zoffs = list(range(0, zr - CHUNK, CHUNK)) + [zr - CHUNK]
    mesh = plsc.VectorSubcoreMesh(core_axis_name="c", subcore_axis_name="s",
                                  num_cores=NC, num_subcores=NS)

    grp = 16
    n_grp = cpt // grp

    @functools.partial(
        pl.kernel, mesh=mesh,
        out_type=jax.ShapeDtypeStruct((NC, np_rows, d), jnp.float32),
        scratch_types=[
            pltpu.VMEM((grp, CHUNK), jnp.int32),
            pltpu.VMEM((grp, CHUNK), jnp.int32),
            pltpu.VMEM((2, CHUNK, d), jnp.float32),
            pltpu.VMEM_SHARED((np_rows, d), jnp.float32),
            pltpu.SemaphoreType.DMA((2,)),
        ],
    )
    def prop_kernel(tab_hbm, rowa_hbm, rowb_hbm, col_hbm, zeros_hbm, out_hbm,
                    idx_v, col_v, gbuf, acc_sh, sem):
        cid = lax.axis_index("c")
        sid = lax.axis_index("s")
        pltpu.sync_copy(zeros_hbm, gbuf.at[0])
        for off in zoffs:
            pltpu.sync_copy(gbuf.at[0], acc_sh.at[pl.ds(sid * zr + off, CHUNK)])
        if split_edges:
            base = (sid * NC + cid) * cpt
        else:
            base = sid * cpt
        plsc.subcore_barrier()

        def group(g, carry):
            gb = base + g * grp

            @pl.when(cid == 0)
            def _():
                pltpu.sync_copy(rowa_hbm.at[pl.ds(gb, grp)], idx_v)

            @pl.when(cid == 1)
            def _():
                pltpu.sync_copy(rowb_hbm.at[pl.ds(gb, grp)], idx_v)

            pltpu.sync_copy(col_hbm.at[pl.ds(gb, grp)], col_v)
            pltpu.async_copy(tab_hbm.at[idx_v.at[0]], gbuf.at[0], sem.at[0])
            for j in range(grp):
                b = j % 2
                pltpu.make_async_copy(tab_hbm.at[idx_v.at[j]],
                                      gbuf.at[b], sem.at[b]).wait()
                if j + 1 < grp:
                    pltpu.async_copy(tab_hbm.at[idx_v.at[j + 1]],
                                     gbuf.at[1 - b], sem.at[1 - b])
                pltpu.sync_copy(gbuf.at[b], acc_sh.at[col_v.at[j]], add=True)
            return carry

        lax.fori_loop(0, n_grp, group, 0)
        plsc.subcore_barrier()
        wb_blocks = [(i * CHUNK, CHUNK) for i in range(zr // CHUNK)]
        if zr % CHUNK:
            wb_blocks.append((zr // CHUNK * CHUNK, zr % CHUNK))
        for off, rows in wb_blocks:
            pltpu.sync_copy(acc_sh.at[pl.ds(sid * zr + off, rows)],
                            gbuf.at[0, pl.ds(0, rows)])
            pltpu.sync_copy(gbuf.at[0, pl.ds(0, rows)],
                            out_hbm.at[cid, pl.ds(sid * zr + off, rows)])

    return prop_kernel


def kernel(x, edge_index, W_lin, b_lin, W_beta, b_beta):
    n, d = x.shape
    e = edge_index.shape[1]
    np_rows = -(-n // (NS * 8)) * (NS * 8)
    ep = -(-e // (NC * NS * CHUNK * 8)) * (NC * NS * CHUNK * 8)
    npad = ep - e
    nph = -(-(np_rows + 4096) // (NS * 8)) * (NS * 8)

    row = edge_index[0]
    col = edge_index[1]
    pad_nodes = n + (jnp.arange(npad, dtype=jnp.int32) % (np_rows - n))
    row_p = jnp.concatenate([row, pad_nodes])
    col_p = jnp.concatenate([col, pad_nodes])
    dump = np_rows + (jnp.arange(ep, dtype=jnp.int32) % 4096)
    self_p = jnp.where(row_p == col_p, row_p, dump)
    ncr = ep // CHUNK
    row2d = row_p.reshape(ncr, CHUNK)
    col2d = col_p.reshape(ncr, CHUNK)
    self2d = self_p.reshape(ncr, CHUNK)
    rowb2d = row2d + np_rows

    ones128 = jnp.ones((CHUNK,), jnp.float32)
    zeros_h = jnp.zeros((CHUNK,), jnp.float32)
    zeros_nd = jnp.zeros((CHUNK, d), jnp.float32)
    x_pad = jnp.concatenate([x, jnp.zeros((np_rows - n, d), x.dtype)])

    hista, histb = _sc_hist(nph, ncr)(row2d, self2d, ones128, zeros_h)
    hrow = hista[:np_rows].reshape(np_rows, 1)
    hself = histb[:np_rows].reshape(np_rows, 1)

    nd = jax.ShapeDtypeStruct((np_rows, d), jnp.float32)
    nd2 = jax.ShapeDtypeStruct((2 * np_rows, d), jnp.float32)
    n1 = jax.ShapeDtypeStruct((np_rows, 1), jnp.float32)
    h = _tc_call(_tc1a_body, nd)(x_pad, W_lin, b_lin.reshape(1, d))
    tab1, a1, a2 = _tc_call(functools.partial(_tc1b_body, np_rows),
                            (nd2, n1, n1))(h, hrow, hself)

    s1 = _sc_prop(np_rows, d, ncr, split_edges=False)(
        tab1, row2d, rowb2d, col2d, zeros_nd)

    k = int(n * 0.75)
    t2, sel = _tc_call(functools.partial(_tc2_body, np_rows, n, k),
                       (nd, n1))(tab1, a1, a2, s1)

    prop_split = _sc_prop(np_rows, d, ncr, split_edges=True)
    s2 = prop_split(t2, row2d, row2d, col2d, zeros_nd)

    t3 = _tc_call(_tc3_body, nd)(
        s1, s2, h, sel,
        W_beta[:, :d], W_beta[:, d:], b_beta.reshape(1, 1))

    s3 = prop_split(t3, row2d, row2d, col2d, zeros_nd)

    out = _tc_call(_tc4_body, nd)(h, s3)
    return out[:n]
